# baseline (speedup 1.0000x reference)
import functools

import jax
import jax.numpy as jnp
from jax import lax
from jax.experimental import pallas as pl
from jax.experimental.pallas import tpu as pltpu
from jax.experimental.pallas import tpu_sc as plsc

B = 16384
GD = 64
MD = 128
NW = 32
BPW = B // NW
CHUNK = 64
NCH = BPW // CHUNK

BS = 1024
NBLK = B // BS

_sc_mesh = plsc.VectorSubcoreMesh(core_axis_name="c", subcore_axis_name="s")


def _pipeline(base, idx_sets, tab_sets, buf_sets, fire_out, drain_out, sems):
    set0, set1 = buf_sets
    gsem0, gsem1, wsem0, wsem1 = sems

    def fire_gather(bufs, j, sem):
        for t, ix, b in zip(tab_sets, idx_sets, bufs):
            pltpu.async_copy(t.at[ix.at[pl.ds(j * CHUNK, CHUNK)]], b, sem)

    def drain_gather(bufs, sem):
        for t, b in zip(tab_sets, bufs):
            pltpu.make_async_copy(t.at[pl.ds(0, CHUNK)], b, sem).wait()

    fire_gather(set0, 0, gsem0)

    def pair(p, carry):
        j0 = 2 * p
        drain_gather(set0, gsem0)

        @pl.when(p > 0)
        def _():
            drain_out(set1, wsem1)

        fire_gather(set1, j0 + 1, gsem1)
        fire_out(set0, j0, wsem0)
        drain_gather(set1, gsem1)
        drain_out(set0, wsem0)

        @pl.when(p + 1 < NCH // 2)
        def _():
            fire_gather(set0, j0 + 2, gsem0)

        fire_out(set1, j0 + 1, wsem1)
        return carry

    lax.fori_loop(0, NCH // 2, pair, 0)
    drain_out(set1, wsem1)


@functools.partial(
    pl.kernel,
    mesh=_sc_mesh,
    compiler_params=pltpu.CompilerParams(use_tc_tiling_on_sc=False),
    out_type=(
        jax.ShapeDtypeStruct((B, MD), jnp.float32),
        jax.ShapeDtypeStruct((B, MD), jnp.float32),
        jax.ShapeDtypeStruct((B, MD), jnp.float32),
        jax.ShapeDtypeStruct((B, MD), jnp.float32),
    ),
    scratch_types=(
        pltpu.VMEM((BPW,), jnp.int32),
        pltpu.VMEM((BPW,), jnp.int32),
        pltpu.VMEM((BPW,), jnp.int32),
        pltpu.VMEM((BPW,), jnp.int32),
        pltpu.VMEM((CHUNK, MD), jnp.float32),
        pltpu.VMEM((CHUNK, MD), jnp.float32),
        pltpu.VMEM((CHUNK, MD), jnp.float32),
        pltpu.VMEM((CHUNK, MD), jnp.float32),
        pltpu.VMEM((CHUNK, MD), jnp.float32),
        pltpu.VMEM((CHUNK, MD), jnp.float32),
        pltpu.VMEM((CHUNK, MD), jnp.float32),
        pltpu.VMEM((CHUNK, MD), jnp.float32),
        pltpu.SemaphoreType.DMA,
        pltpu.SemaphoreType.DMA,
        pltpu.SemaphoreType.DMA,
        pltpu.SemaphoreType.DMA,
    ),
)
def _sc_gather_mlp(uix_h, iix_h, a1x_h, a2x_h,
                   um_t, im_t, a1_t, a2_t,
                   um_o, im_o, a1_o, a2_o,
                   uix_v, iix_v, a1x_v, a2x_v,
                   um_b0, im_b0, a1_b0, a2_b0,
                   um_b1, im_b1, a1_b1, a2_b1,
                   gsem0, gsem1, wsem0, wsem1):
    wid = lax.axis_index("s") * 2 + lax.axis_index("c")
    base = wid * BPW
    pltpu.sync_copy(uix_h.at[pl.ds(base, BPW)], uix_v)
    pltpu.sync_copy(iix_h.at[pl.ds(base, BPW)], iix_v)
    pltpu.sync_copy(a1x_h.at[pl.ds(base, BPW)], a1x_v)
    pltpu.sync_copy(a2x_h.at[pl.ds(base, BPW)], a2x_v)

    outs = (um_o, im_o, a1_o, a2_o)

    def fire_out(bufs, j, sem):
        off = base + j * CHUNK
        for b, o in zip(bufs, outs):
            pltpu.async_copy(b, o.at[pl.ds(off, CHUNK)], sem)

    def drain_out(bufs, sem):
        for b, o in zip(bufs, outs):
            pltpu.make_async_copy(b, o.at[pl.ds(base, CHUNK)], sem).wait()

    _pipeline(base,
              (uix_v, iix_v, a1x_v, a2x_v),
              (um_t, im_t, a1_t, a2_t),
              ((um_b0, im_b0, a1_b0, a2_b0), (um_b1, im_b1, a1_b1, a2_b1)),
              fire_out, drain_out,
              (gsem0, gsem1, wsem0, wsem1))


@functools.partial(
    pl.kernel,
    mesh=_sc_mesh,
    compiler_params=pltpu.CompilerParams(use_tc_tiling_on_sc=False),
    out_type=jax.ShapeDtypeStruct((B, 2 * GD), jnp.bfloat16),
    scratch_types=(
        pltpu.VMEM((BPW,), jnp.int32),
        pltpu.VMEM((BPW,), jnp.int32),
        pltpu.VMEM((CHUNK, 2 * GD), jnp.bfloat16),
        pltpu.VMEM((CHUNK, 2 * GD), jnp.bfloat16),
        pltpu.VMEM((CHUNK, 2 * GD), jnp.bfloat16),
        pltpu.VMEM((CHUNK, 2 * GD), jnp.bfloat16),
        pltpu.SemaphoreType.DMA,
        pltpu.SemaphoreType.DMA,
        pltpu.SemaphoreType.DMA,
        pltpu.SemaphoreType.DMA,
    ),
)
def _sc_gather_gmf(uix_h, iix_h, gc_t, gg_o,
                   uix_v, iix_v,
                   ug_b0, ig_b0, ug_b1, ig_b1,
                   gsem0, gsem1, wsem0, wsem1):
    wid = lax.axis_index("s") * 2 + lax.axis_index("c")
    base = wid * BPW
    pltpu.sync_copy(uix_h.at[pl.ds(base, BPW)], uix_v)
    pltpu.sync_copy(iix_h.at[pl.ds(base, BPW)], iix_v)

    def fire_out(bufs, j, sem):
        off = base + j * CHUNK
        pltpu.async_copy(bufs[0].at[pl.ds(0, CHUNK), pl.ds(0, GD)],
                         gg_o.at[pl.ds(off, CHUNK), pl.ds(0, GD)], sem)
        pltpu.async_copy(bufs[1].at[pl.ds(0, CHUNK), pl.ds(GD, GD)],
                         gg_o.at[pl.ds(off, CHUNK), pl.ds(GD, GD)], sem)

    def drain_out(bufs, sem):
        pltpu.make_async_copy(
            bufs[0].at[pl.ds(0, CHUNK), pl.ds(0, GD)],
            gg_o.at[pl.ds(base, CHUNK), pl.ds(0, GD)], sem).wait()
        pltpu.make_async_copy(
            bufs[1].at[pl.ds(0, CHUNK), pl.ds(GD, GD)],
            gg_o.at[pl.ds(base, CHUNK), pl.ds(GD, GD)], sem).wait()

    _pipeline(base,
              (uix_v, iix_v),
              (gc_t, gc_t),
              ((ug_b0, ig_b0), (ug_b1, ig_b1)),
              fire_out, drain_out,
              (gsem0, gsem1, wsem0, wsem1))


def _tc_body(gg, um, im, a1, a2,
             w0, b0, w1, b1, w2, b2, w3, b3, gw, ow, cc, out):
    bf = jnp.bfloat16
    h = jnp.concatenate([um[...], im[...], a1[...], a2[...]],
                        axis=1).astype(bf)
    h = jnp.maximum(jnp.dot(h, w0[...], preferred_element_type=jnp.float32)
                    + b0[...], 0.0).astype(bf)
    h = jnp.maximum(jnp.dot(h, w1[...], preferred_element_type=jnp.float32)
                    + b1[...], 0.0).astype(bf)
    h = jnp.maximum(jnp.dot(h, w2[...], preferred_element_type=jnp.float32)
                    + b2[...], 0.0).astype(bf)
    h = jnp.maximum(jnp.dot(h, w3[...], preferred_element_type=jnp.float32)
                    + b3[...], 0.0)
    gv = gg[...].astype(jnp.float32)
    gmf = gv[:, :GD] * gv[:, GD:]
    z = (jnp.sum(gmf * gw[...], axis=1)
         + jnp.sum(h * ow[...], axis=1) + cc[0, 0])
    out[...] = (100.0 / (1.0 + jnp.exp(-z))).reshape(1, 1, BS)


def _tc_mlp(gg, um, im, a1, a2, w0t, b0, w1t, b1, w2t, b2, w3t, b3,
            gw, ow, cc):
    full = lambda shape: pl.BlockSpec(shape, lambda i: (0, 0))
    blk = lambda d: pl.BlockSpec((BS, d), lambda i: (i, 0))
    return pl.pallas_call(
        _tc_body,
        grid=(NBLK,),
        in_specs=[
            blk(2 * GD), blk(MD), blk(MD), blk(MD), blk(MD),
            full((4 * MD, 256)), full((1, 256)),
            full((256, 128)), full((1, 128)),
            full((128, 64)), full((1, 64)),
            full((64, 32)), full((1, 32)),
            full((1, GD)), full((1, 32)), full((1, 1)),
        ],
        out_specs=pl.BlockSpec((1, 1, BS), lambda i: (i, 0, 0)),
        out_shape=jax.ShapeDtypeStruct((NBLK, 1, BS), jnp.float32),
    )(gg, um, im, a1, a2, w0t, b0, w1t, b1, w2t, b2, w3t, b3, gw, ow, cc)


def kernel(user_idx, item_idx, item_attr1_idx, item_attr2_idx,
           user_gmf_w, item_gmf_w, user_mlp_w, item_mlp_w, attr1_w, attr2_w,
           gmf_out_w, gmf_out_b,
           mlp_w0, mlp_b0, mlp_w1, mlp_b1, mlp_w2, mlp_b2, mlp_w3, mlp_b3,
           mlp_out_w, mlp_out_b, fusion_w, fusion_b):
    uix = user_idx.astype(jnp.int32)
    iix = item_idx.astype(jnp.int32)
    x1 = item_attr1_idx.astype(jnp.int32)
    x2 = item_attr2_idx.astype(jnp.int32)

    um, im, a1, a2 = _sc_gather_mlp(
        uix, iix, x1, x2, user_mlp_w, item_mlp_w, attr1_w, attr2_w)
    bfc = jnp.bfloat16
    gcat = jnp.concatenate([user_gmf_w.astype(bfc),
                            item_gmf_w.astype(bfc)], axis=1)
    gg = _sc_gather_gmf(uix, iix, gcat)

    f0 = fusion_w[0, 0]
    f1 = fusion_w[0, 1]
    gw = gmf_out_w * f0
    ow = mlp_out_w * f1
    cc = (f0 * gmf_out_b[0] + f1 * mlp_out_b[0] + fusion_b[0]).reshape(1, 1)

    bf = jnp.bfloat16
    out = _tc_mlp(gg, um, im, a1, a2,
                  mlp_w0.T.astype(bf), mlp_b0.reshape(1, -1),
                  mlp_w1.T.astype(bf), mlp_b1.reshape(1, -1),
                  mlp_w2.T.astype(bf), mlp_b2.reshape(1, -1),
                  mlp_w3.T.astype(bf), mlp_b3.reshape(1, -1),
                  gw, ow, cc)
    return out.reshape(B)


# speedup vs baseline: 1.4720x; 1.4720x over previous
import functools

import jax
import jax.numpy as jnp
from jax import lax
from jax.experimental import pallas as pl
from jax.experimental.pallas import tpu as pltpu
from jax.experimental.pallas import tpu_sc as plsc

B = 16384
GD = 64
MD = 128
NW = 32
BPW = B // NW
CHUNK = 64
NCH = BPW // CHUNK

BS = 1024
NBLK = B // BS

_sc_mesh = plsc.VectorSubcoreMesh(core_axis_name="c", subcore_axis_name="s")


def _pipeline(base, idx_sets, tab_sets, buf_sets, fire_out, drain_out, sems):
    set0, set1 = buf_sets
    gsem0, gsem1, wsem0, wsem1 = sems

    def fire_gather(bufs, j, sem):
        for t, ix, b in zip(tab_sets, idx_sets, bufs):
            pltpu.async_copy(t.at[ix.at[pl.ds(j * CHUNK, CHUNK)]], b, sem)

    def drain_gather(bufs, sem):
        for t, b in zip(tab_sets, bufs):
            pltpu.make_async_copy(t.at[pl.ds(0, CHUNK)], b, sem).wait()

    fire_gather(set0, 0, gsem0)

    def pair(p, carry):
        j0 = 2 * p
        drain_gather(set0, gsem0)

        @pl.when(p > 0)
        def _():
            drain_out(set1, wsem1)

        fire_gather(set1, j0 + 1, gsem1)
        fire_out(set0, j0, wsem0)
        drain_gather(set1, gsem1)
        drain_out(set0, wsem0)

        @pl.when(p + 1 < NCH // 2)
        def _():
            fire_gather(set0, j0 + 2, gsem0)

        fire_out(set1, j0 + 1, wsem1)
        return carry

    lax.fori_loop(0, NCH // 2, pair, 0)
    drain_out(set1, wsem1)


@functools.partial(
    pl.kernel,
    mesh=_sc_mesh,
    compiler_params=pltpu.CompilerParams(use_tc_tiling_on_sc=False),
    out_type=(
        jax.ShapeDtypeStruct((B, MD), jnp.float32),
        jax.ShapeDtypeStruct((B, MD), jnp.float32),
        jax.ShapeDtypeStruct((B, MD), jnp.float32),
        jax.ShapeDtypeStruct((B, MD), jnp.float32),
    ),
    scratch_types=(
        pltpu.VMEM((BPW,), jnp.int32),
        pltpu.VMEM((BPW,), jnp.int32),
        pltpu.VMEM((BPW,), jnp.int32),
        pltpu.VMEM((BPW,), jnp.int32),
        pltpu.VMEM((CHUNK, MD), jnp.float32),
        pltpu.VMEM((CHUNK, MD), jnp.float32),
        pltpu.VMEM((CHUNK, MD), jnp.float32),
        pltpu.VMEM((CHUNK, MD), jnp.float32),
        pltpu.VMEM((CHUNK, MD), jnp.float32),
        pltpu.VMEM((CHUNK, MD), jnp.float32),
        pltpu.VMEM((CHUNK, MD), jnp.float32),
        pltpu.VMEM((CHUNK, MD), jnp.float32),
        pltpu.SemaphoreType.DMA,
        pltpu.SemaphoreType.DMA,
        pltpu.SemaphoreType.DMA,
        pltpu.SemaphoreType.DMA,
    ),
)
def _sc_gather_mlp(uix_h, iix_h, a1x_h, a2x_h,
                   um_t, im_t, a1_t, a2_t,
                   um_o, im_o, a1_o, a2_o,
                   uix_v, iix_v, a1x_v, a2x_v,
                   um_b0, im_b0, a1_b0, a2_b0,
                   um_b1, im_b1, a1_b1, a2_b1,
                   gsem0, gsem1, wsem0, wsem1):
    wid = lax.axis_index("s") * 2 + lax.axis_index("c")
    base = wid * BPW
    pltpu.sync_copy(uix_h.at[pl.ds(base, BPW)], uix_v)
    pltpu.sync_copy(iix_h.at[pl.ds(base, BPW)], iix_v)
    pltpu.sync_copy(a1x_h.at[pl.ds(base, BPW)], a1x_v)
    pltpu.sync_copy(a2x_h.at[pl.ds(base, BPW)], a2x_v)

    outs = (um_o, im_o, a1_o, a2_o)

    def fire_out(bufs, j, sem):
        off = base + j * CHUNK
        for b, o in zip(bufs, outs):
            pltpu.async_copy(b, o.at[pl.ds(off, CHUNK)], sem)

    def drain_out(bufs, sem):
        for b, o in zip(bufs, outs):
            pltpu.make_async_copy(b, o.at[pl.ds(base, CHUNK)], sem).wait()

    _pipeline(base,
              (uix_v, iix_v, a1x_v, a2x_v),
              (um_t, im_t, a1_t, a2_t),
              ((um_b0, im_b0, a1_b0, a2_b0), (um_b1, im_b1, a1_b1, a2_b1)),
              fire_out, drain_out,
              (gsem0, gsem1, wsem0, wsem1))


@functools.partial(
    pl.kernel,
    mesh=_sc_mesh,
    compiler_params=pltpu.CompilerParams(use_tc_tiling_on_sc=False),
    out_type=jax.ShapeDtypeStruct((B, 2 * GD), jnp.float32),
    scratch_types=(
        pltpu.VMEM((BPW,), jnp.int32),
        pltpu.VMEM((BPW,), jnp.int32),
        pltpu.VMEM((CHUNK, 2 * GD), jnp.float32),
        pltpu.VMEM((CHUNK, 2 * GD), jnp.float32),
        pltpu.VMEM((CHUNK, 2 * GD), jnp.float32),
        pltpu.VMEM((CHUNK, 2 * GD), jnp.float32),
        pltpu.SemaphoreType.DMA,
        pltpu.SemaphoreType.DMA,
        pltpu.SemaphoreType.DMA,
        pltpu.SemaphoreType.DMA,
    ),
)
def _sc_gather_gmf(uix_h, iix_h, gc_t, gg_o,
                   uix_v, iix_v,
                   ug_b0, ig_b0, ug_b1, ig_b1,
                   gsem0, gsem1, wsem0, wsem1):
    wid = lax.axis_index("s") * 2 + lax.axis_index("c")
    base = wid * BPW
    pltpu.sync_copy(uix_h.at[pl.ds(base, BPW)], uix_v)
    pltpu.sync_copy(iix_h.at[pl.ds(base, BPW)], iix_v)

    def fire_out(bufs, j, sem):
        off = base + j * CHUNK
        pltpu.async_copy(bufs[0].at[pl.ds(0, CHUNK), pl.ds(0, GD)],
                         gg_o.at[pl.ds(off, CHUNK), pl.ds(0, GD)], sem)
        pltpu.async_copy(bufs[1].at[pl.ds(0, CHUNK), pl.ds(GD, GD)],
                         gg_o.at[pl.ds(off, CHUNK), pl.ds(GD, GD)], sem)

    def drain_out(bufs, sem):
        pltpu.make_async_copy(
            bufs[0].at[pl.ds(0, CHUNK), pl.ds(0, GD)],
            gg_o.at[pl.ds(base, CHUNK), pl.ds(0, GD)], sem).wait()
        pltpu.make_async_copy(
            bufs[1].at[pl.ds(0, CHUNK), pl.ds(GD, GD)],
            gg_o.at[pl.ds(base, CHUNK), pl.ds(GD, GD)], sem).wait()

    _pipeline(base,
              (uix_v, iix_v),
              (gc_t, gc_t),
              ((ug_b0, ig_b0), (ug_b1, ig_b1)),
              fire_out, drain_out,
              (gsem0, gsem1, wsem0, wsem1))


def _tc_body(gg, um, im, a1, a2,
             w0, b0, w1, b1, w2, b2, w3, b3, gw, ow, cc, out):
    bf = jnp.bfloat16
    h = jnp.concatenate([um[...], im[...], a1[...], a2[...]],
                        axis=1).astype(bf)
    h = jnp.maximum(jnp.dot(h, w0[...], preferred_element_type=jnp.float32)
                    + b0[...], 0.0).astype(bf)
    h = jnp.maximum(jnp.dot(h, w1[...], preferred_element_type=jnp.float32)
                    + b1[...], 0.0).astype(bf)
    h = jnp.maximum(jnp.dot(h, w2[...], preferred_element_type=jnp.float32)
                    + b2[...], 0.0).astype(bf)
    h = jnp.maximum(jnp.dot(h, w3[...], preferred_element_type=jnp.float32)
                    + b3[...], 0.0)
    gv = gg[...]
    gmf = gv[:, :GD] * gv[:, GD:]
    z = (jnp.sum(gmf * gw[...], axis=1)
         + jnp.sum(h * ow[...], axis=1) + cc[0, 0])
    out[...] = (100.0 / (1.0 + jnp.exp(-z))).reshape(1, 1, BS)


def _tc_mlp(gg, um, im, a1, a2, w0t, b0, w1t, b1, w2t, b2, w3t, b3,
            gw, ow, cc):
    full = lambda shape: pl.BlockSpec(shape, lambda i: (0, 0))
    blk = lambda d: pl.BlockSpec((BS, d), lambda i: (i, 0))
    return pl.pallas_call(
        _tc_body,
        grid=(NBLK,),
        in_specs=[
            blk(2 * GD), blk(MD), blk(MD), blk(MD), blk(MD),
            full((4 * MD, 256)), full((1, 256)),
            full((256, 128)), full((1, 128)),
            full((128, 64)), full((1, 64)),
            full((64, 32)), full((1, 32)),
            full((1, GD)), full((1, 32)), full((1, 1)),
        ],
        out_specs=pl.BlockSpec((1, 1, BS), lambda i: (i, 0, 0)),
        out_shape=jax.ShapeDtypeStruct((NBLK, 1, BS), jnp.float32),
    )(gg, um, im, a1, a2, w0t, b0, w1t, b1, w2t, b2, w3t, b3, gw, ow, cc)


def kernel(user_idx, item_idx, item_attr1_idx, item_attr2_idx,
           user_gmf_w, item_gmf_w, user_mlp_w, item_mlp_w, attr1_w, attr2_w,
           gmf_out_w, gmf_out_b,
           mlp_w0, mlp_b0, mlp_w1, mlp_b1, mlp_w2, mlp_b2, mlp_w3, mlp_b3,
           mlp_out_w, mlp_out_b, fusion_w, fusion_b):
    uix = user_idx.astype(jnp.int32)
    iix = item_idx.astype(jnp.int32)
    x1 = item_attr1_idx.astype(jnp.int32)
    x2 = item_attr2_idx.astype(jnp.int32)

    um, im, a1, a2 = _sc_gather_mlp(
        uix, iix, x1, x2, user_mlp_w, item_mlp_w, attr1_w, attr2_w)
    gcat = jnp.concatenate([user_gmf_w, item_gmf_w], axis=1)
    gg = _sc_gather_gmf(uix, iix, gcat)

    f0 = fusion_w[0, 0]
    f1 = fusion_w[0, 1]
    gw = gmf_out_w * f0
    ow = mlp_out_w * f1
    cc = (f0 * gmf_out_b[0] + f1 * mlp_out_b[0] + fusion_b[0]).reshape(1, 1)

    bf = jnp.bfloat16
    out = _tc_mlp(gg, um, im, a1, a2,
                  mlp_w0.T.astype(bf), mlp_b0.reshape(1, -1),
                  mlp_w1.T.astype(bf), mlp_b1.reshape(1, -1),
                  mlp_w2.T.astype(bf), mlp_b2.reshape(1, -1),
                  mlp_w3.T.astype(bf), mlp_b3.reshape(1, -1),
                  gw, ow, cc)
    return out.reshape(B)
